# baseline (speedup 1.0000x reference)
import functools

import jax
import jax.numpy as jnp
from jax import lax
from jax.experimental import pallas as pl
from jax.experimental.pallas import tpu as pltpu
from jax.experimental.pallas import tpu_sc as plsc

_B, _T, _C = 32, 1025, 768
_K = _T - 256
_PPAD = 784
_CHUNK = 64
_NFULL = _K // _CHUNK


def _topk_body(r_ref, x_ref, idx_ref):
    xb = x_ref[0]
    rs = []
    for c in range(_C // 128):
        ch = xb[:, c * 128:(c + 1) * 128]
        rs.append(jnp.sum(ch * ch, axis=-1))
    q = rs[0]
    for c in range(1, _C // 128):
        q = q + rs[c]
    s = jnp.sqrt(q)

    ii1 = lax.broadcasted_iota(jnp.int32, (_T, 1), 0)
    jj1 = lax.broadcasted_iota(jnp.int32, (1, _T), 1)
    sc = jnp.where(ii1 == 0, jnp.inf, s[:, None])
    sr = jnp.where(jj1 == 0, jnp.inf, s[None, :])

    ii = lax.broadcasted_iota(jnp.int32, (_T, _T), 0)
    jj = lax.broadcasted_iota(jnp.int32, (_T, _T), 1)
    beats = jnp.where((sr > sc) | ((sr == sc) & (jj < ii)), 1.0, 0.0)
    rank = jnp.sum(beats, axis=1, keepdims=True)
    rank32 = rank.astype(jnp.int32)

    pr = lax.broadcasted_iota(jnp.int32, (_T, _PPAD), 1)
    it = lax.broadcasted_iota(jnp.int32, (_T, _PPAD), 0)
    loc = jnp.sum(jnp.where(rank32 == pr, it, 0), axis=0)
    idx_ref[0, 0, :] = loc + (r_ref[0] - 256)


_topk_call = pl.pallas_call(
    _topk_body,
    grid=(_B,),
    in_specs=[
        pl.BlockSpec(memory_space=pltpu.SMEM),
        pl.BlockSpec((1, _T, _C), lambda b: (b, 0, 0)),
    ],
    out_specs=pl.BlockSpec((1, 1, _PPAD), lambda b: (b, 0, 0)),
    out_shape=jax.ShapeDtypeStruct((_B, 1, _PPAD), jnp.int32),
)


def _gather_body(x, idxp, out, idx_v, buf0, buf1, tail_v, g0, g1, s0, s1):
    wid = lax.axis_index("s") * 2 + lax.axis_index("c")
    pltpu.sync_copy(idxp.at[wid], idx_v)
    xw = x.at[wid]
    ow = out.at[wid]
    bufs = (buf0, buf1)
    gsem = (g0, g1)
    ssem = (s0, s1)

    def start_gather(c):
        if c == _NFULL:
            return pltpu.async_copy(xw.at[idx_v.at[0, pl.ds(_NFULL * _CHUNK, 1)]],
                                    tail_v, gsem[c % 2])
        return pltpu.async_copy(xw.at[idx_v.at[0, pl.ds(c * _CHUNK, _CHUNK)]],
                                bufs[c % 2], gsem[c % 2])

    def start_store(c):
        if c == _NFULL:
            return pltpu.async_copy(tail_v, ow.at[pl.ds(_NFULL * _CHUNK, 1)],
                                    ssem[c % 2])
        return pltpu.async_copy(bufs[c % 2], ow.at[pl.ds(c * _CHUNK, _CHUNK)],
                                ssem[c % 2])

    nch = _NFULL + 1
    g_h = [None] * nch
    s_h = [None] * nch
    g_h[0] = start_gather(0)
    for c in range(nch):
        if c + 1 < nch:
            if c - 1 >= 0 and c + 1 < _NFULL:
                s_h[c - 1].wait()
            g_h[c + 1] = start_gather(c + 1)
        g_h[c].wait()
        s_h[c] = start_store(c)
    s_h[nch - 3].wait()
    s_h[nch - 2].wait()
    s_h[nch - 1].wait()


@functools.lru_cache(maxsize=1)
def _make_gather_call():
    return functools.partial(
        pl.kernel,
        out_type=jax.ShapeDtypeStruct((_B, _K, _C), jnp.float32),
        mesh=plsc.VectorSubcoreMesh(core_axis_name="c", subcore_axis_name="s"),
        compiler_params=pltpu.CompilerParams(use_tc_tiling_on_sc=True),
        scratch_types=[
            pltpu.VMEM((1, _PPAD), jnp.int32),
            pltpu.VMEM((_CHUNK, _C), jnp.float32),
            pltpu.VMEM((_CHUNK, _C), jnp.float32),
            pltpu.VMEM((1, _C), jnp.float32),
            pltpu.SemaphoreType.DMA,
            pltpu.SemaphoreType.DMA,
            pltpu.SemaphoreType.DMA,
            pltpu.SemaphoreType.DMA,
        ],
    )(_gather_body)


def kernel(x, layer_idx, requested_r):
    del layer_idx
    r_arr = jnp.asarray(requested_r, jnp.int32).reshape(1)
    idxp = _topk_call(r_arr, x)
    return _make_gather_call()(x, idxp)


# speedup vs baseline: 1.0048x; 1.0048x over previous
import functools

import jax
import jax.numpy as jnp
from jax import lax
from jax.experimental import pallas as pl
from jax.experimental.pallas import tpu as pltpu
from jax.experimental.pallas import tpu_sc as plsc

_B, _T, _C = 32, 1025, 768
_K = _T - 256
_TPAD = 1040
_CHUNK = 64
_NFULL = _K // _CHUNK
_L = 16


def _topk_body(x_ref, rank_ref):
    xb = x_ref[0]
    rs = []
    for c in range(_C // 128):
        ch = xb[:, c * 128:(c + 1) * 128]
        rs.append(jnp.sum(ch * ch, axis=-1))
    q = rs[0]
    for c in range(1, _C // 128):
        q = q + rs[c]
    s = jnp.sqrt(q)

    ii1 = lax.broadcasted_iota(jnp.int32, (_T, 1), 0)
    jj1 = lax.broadcasted_iota(jnp.int32, (1, _T), 1)
    sj = jnp.where(ii1 == 0, jnp.inf, s[:, None])
    si = jnp.where(jj1 == 0, jnp.inf, s[None, :])

    jj = lax.broadcasted_iota(jnp.int32, (_T, _T), 0)
    ii = lax.broadcasted_iota(jnp.int32, (_T, _T), 1)
    beats = jnp.where((sj > si) | ((sj == si) & (jj < ii)), 1.0, 0.0)
    rank = jnp.sum(beats, axis=0)
    rank_ref[0, 0, :_T] = rank.astype(jnp.int32)


_topk_call = pl.pallas_call(
    _topk_body,
    grid=(_B,),
    in_specs=[pl.BlockSpec((1, _T, _C), lambda b: (b, 0, 0))],
    out_specs=pl.BlockSpec((1, 1, _TPAD), lambda b: (b, 0, 0)),
    out_shape=jax.ShapeDtypeStruct((_B, 1, _TPAD), jnp.int32),
)


def _gather_body(x, rankp, shp, out, rank_v, idx_v, sh_v, buf0, buf1, tail_v,
                 g0, g1, s0, s1):
    wid = lax.axis_index("s") * 2 + lax.axis_index("c")
    pltpu.sync_copy(rankp.at[wid], rank_v)
    pltpu.sync_copy(shp, sh_v)
    xw = x.at[wid]
    ow = out.at[wid]

    sh = sh_v[...]
    for k in range(_TPAD // _L):
        r16 = rank_v[0, pl.ds(k * _L, _L)]
        t16 = lax.iota(jnp.int32, _L) + (k * _L)
        keep = (r16 < _K) & (t16 <= _T - 1)
        plsc.store_scatter(idx_v, [r16], t16 + sh, mask=keep)

    bufs = (buf0, buf1)
    gsem = (g0, g1)
    ssem = (s0, s1)

    def start_gather(c):
        if c == _NFULL:
            return pltpu.async_copy(xw.at[idx_v.at[pl.ds(_NFULL * _CHUNK, 1)]],
                                    tail_v, gsem[c % 2])
        return pltpu.async_copy(xw.at[idx_v.at[pl.ds(c * _CHUNK, _CHUNK)]],
                                bufs[c % 2], gsem[c % 2])

    def start_store(c):
        if c == _NFULL:
            return pltpu.async_copy(tail_v, ow.at[pl.ds(_NFULL * _CHUNK, 1)],
                                    ssem[c % 2])
        return pltpu.async_copy(bufs[c % 2], ow.at[pl.ds(c * _CHUNK, _CHUNK)],
                                ssem[c % 2])

    nch = _NFULL + 1
    g_h = [None] * nch
    s_h = [None] * nch
    g_h[0] = start_gather(0)
    for c in range(nch):
        if c + 1 < nch:
            if c - 1 >= 0 and c + 1 < _NFULL:
                s_h[c - 1].wait()
            g_h[c + 1] = start_gather(c + 1)
        g_h[c].wait()
        s_h[c] = start_store(c)
    s_h[nch - 3].wait()
    s_h[nch - 2].wait()
    s_h[nch - 1].wait()


@functools.lru_cache(maxsize=1)
def _make_gather_call():
    return functools.partial(
        pl.kernel,
        out_type=jax.ShapeDtypeStruct((_B, _K, _C), jnp.float32),
        mesh=plsc.VectorSubcoreMesh(core_axis_name="c", subcore_axis_name="s"),
        compiler_params=pltpu.CompilerParams(needs_layout_passes=False),
        scratch_types=[
            pltpu.VMEM((1, _TPAD), jnp.int32),
            pltpu.VMEM((_K - 1 + _L,), jnp.int32),
            pltpu.VMEM((_L,), jnp.int32),
            pltpu.VMEM((_CHUNK, _C), jnp.float32),
            pltpu.VMEM((_CHUNK, _C), jnp.float32),
            pltpu.VMEM((1, _C), jnp.float32),
            pltpu.SemaphoreType.DMA,
            pltpu.SemaphoreType.DMA,
            pltpu.SemaphoreType.DMA,
            pltpu.SemaphoreType.DMA,
        ],
    )(_gather_body)


def kernel(x, layer_idx, requested_r):
    del layer_idx
    rankp = _topk_call(x)
    shp = jnp.full((_L,), requested_r - 256, jnp.int32)
    return _make_gather_call()(x, rankp, shp)


# speedup vs baseline: 1.2763x; 1.2702x over previous
import functools

import jax
import jax.numpy as jnp
from jax import lax
from jax.experimental import pallas as pl
from jax.experimental.pallas import tpu as pltpu
from jax.experimental.pallas import tpu_sc as plsc

_B, _T, _C = 32, 1025, 768
_K = _T - 256
_TPAD = 1040
_CHUNK = 64
_NFULL = _K // _CHUNK
_L = 16


def _topk_body(x_ref, rank_ref):
    xb = x_ref[0]
    rs = []
    for c in range(_C // 128):
        ch = xb[:, c * 128:(c + 1) * 128]
        rs.append(jnp.sum(ch * ch, axis=-1))
    q = rs[0]
    for c in range(1, _C // 128):
        q = q + rs[c]
    s = jnp.sqrt(q)

    ii1 = lax.broadcasted_iota(jnp.int32, (_T, 1), 0)
    jj1 = lax.broadcasted_iota(jnp.int32, (1, _T), 1)
    sj = jnp.where(ii1 == 0, jnp.inf, s[:, None])
    si = jnp.where(jj1 == 0, jnp.inf, s[None, :])

    jj = lax.broadcasted_iota(jnp.int32, (_T, _T), 0)
    ii = lax.broadcasted_iota(jnp.int32, (_T, _T), 1)
    beats = jnp.where((sj > si) | ((sj == si) & (jj < ii)), 1.0, 0.0)
    rank = jnp.sum(beats, axis=0)
    rank_ref[0, 0, :_T] = rank.astype(jnp.int32)


_topk_call = pl.pallas_call(
    _topk_body,
    grid=(_B,),
    in_specs=[pl.BlockSpec((1, _T, _C), lambda b: (b, 0, 0))],
    out_specs=pl.BlockSpec((1, 1, _TPAD), lambda b: (b, 0, 0)),
    out_shape=jax.ShapeDtypeStruct((_B, 1, _TPAD), jnp.int32),
)


def _chunk_base(c):
    return c * _CHUNK if c < _NFULL else _K - _CHUNK


def _gather_body(x, rankp, shp, out, rank_v, idx_v, oidx_v, sh_v, buf0, buf1,
                 g0, g1, s0, s1):
    wid = lax.axis_index("s") * 2 + lax.axis_index("c")
    pltpu.sync_copy(rankp.at[wid], rank_v)
    pltpu.sync_copy(shp, sh_v)
    xw = x.at[wid]

    sh = sh_v[...]
    for k in range(_TPAD // _L):
        r16 = rank_v[0, pl.ds(k * _L, _L)]
        t16 = lax.iota(jnp.int32, _L) + (k * _L)
        tv = t16 + sh
        tok_ok = t16 <= _T - 1
        plsc.store_scatter(idx_v, [r16], tv,
                           mask=(r16 < _NFULL * _CHUNK) & tok_ok)
        plsc.store_scatter(idx_v, [r16 + (_NFULL + 1) * _CHUNK - _K], tv,
                           mask=(r16 >= _K - _CHUNK) & (r16 < _K) & tok_ok)

    nch = _NFULL + 1
    for c in range(nch):
        for v in range(_CHUNK // _L):
            p16 = lax.iota(jnp.int32, _L) + (_chunk_base(c) + v * _L)
            oidx_v[c, pl.ds(v * _L, _L)] = p16 * _B + wid

    bufs = (buf0, buf1)
    gsem = (g0, g1)
    ssem = (s0, s1)

    def start_gather(c):
        return pltpu.async_copy(xw.at[idx_v.at[pl.ds(c * _CHUNK, _CHUNK)]],
                                bufs[c % 2], gsem[c % 2])

    def start_store(c):
        return pltpu.async_copy(bufs[c % 2], out.at[oidx_v.at[c]], ssem[c % 2])

    g_h = [None] * nch
    s_h = [None] * nch
    g_h[0] = start_gather(0)
    for c in range(nch):
        if c + 1 < nch:
            if c - 1 >= 0:
                s_h[c - 1].wait()
            g_h[c + 1] = start_gather(c + 1)
        g_h[c].wait()
        s_h[c] = start_store(c)
    s_h[nch - 2].wait()
    s_h[nch - 1].wait()


@functools.lru_cache(maxsize=1)
def _make_gather_call():
    return functools.partial(
        pl.kernel,
        out_type=jax.ShapeDtypeStruct((_B * _K, _C), jnp.float32),
        mesh=plsc.VectorSubcoreMesh(core_axis_name="c", subcore_axis_name="s"),
        compiler_params=pltpu.CompilerParams(needs_layout_passes=False),
        scratch_types=[
            pltpu.VMEM((1, _TPAD), jnp.int32),
            pltpu.VMEM(((_NFULL + 1) * _CHUNK,), jnp.int32),
            pltpu.VMEM((_NFULL + 1, _CHUNK), jnp.int32),
            pltpu.VMEM((_L,), jnp.int32),
            pltpu.VMEM((_CHUNK, _C), jnp.float32),
            pltpu.VMEM((_CHUNK, _C), jnp.float32),
            pltpu.SemaphoreType.DMA,
            pltpu.SemaphoreType.DMA,
            pltpu.SemaphoreType.DMA,
            pltpu.SemaphoreType.DMA,
        ],
    )(_gather_body)


def kernel(x, layer_idx, requested_r):
    del layer_idx
    rankp = _topk_call(x)
    shp = jnp.full((_L,), requested_r - 256, jnp.int32)
    outflat = _make_gather_call()(x, rankp, shp)
    return jnp.transpose(outflat.reshape(_K, _B, _C), (1, 0, 2))


# speedup vs baseline: 1.6899x; 1.3241x over previous
import functools

import jax
import jax.numpy as jnp
from jax import lax
from jax.experimental import pallas as pl
from jax.experimental.pallas import tpu as pltpu
from jax.experimental.pallas import tpu_sc as plsc

_B, _T, _C = 32, 1025, 768
_K = _T - 256
_TPAD = 1040
_CHUNK = 64
_NFULL = _K // _CHUNK
_L = 16


_TBLK = 41


def _scores_body(xt_ref, s_ref):
    xb = xt_ref[...]
    rs = []
    for c in range(_C // 128):
        ch = xb[:, :, c * 128:(c + 1) * 128]
        rs.append(jnp.sum(ch * ch, axis=-1))
    q = rs[0]
    for c in range(1, _C // 128):
        q = q + rs[c]
    s_ref[0] = jnp.sqrt(q)


_scores_call = pl.pallas_call(
    _scores_body,
    grid=(_T // _TBLK,),
    in_specs=[pl.BlockSpec((_TBLK, _B, _C), lambda i: (i, 0, 0))],
    out_specs=pl.BlockSpec((1, _TBLK, _B), lambda i: (i, 0, 0)),
    out_shape=jax.ShapeDtypeStruct((_T // _TBLK, _TBLK, _B), jnp.float32),
)


def _topk_body(s_ref, rank_ref):
    s = s_ref[0, 0, :]

    ii1 = lax.broadcasted_iota(jnp.int32, (_T, 1), 0)
    jj1 = lax.broadcasted_iota(jnp.int32, (1, _T), 1)
    sj = jnp.where(ii1 == 0, jnp.inf, s[:, None])
    si = jnp.where(jj1 == 0, jnp.inf, s[None, :])

    jj = lax.broadcasted_iota(jnp.int32, (_T, _T), 0)
    ii = lax.broadcasted_iota(jnp.int32, (_T, _T), 1)
    beats = jnp.where((sj > si) | ((sj == si) & (jj < ii)), 1.0, 0.0)
    rank = jnp.sum(beats, axis=0)
    rank_ref[0, 0, :_T] = rank.astype(jnp.int32)


_topk_call = pl.pallas_call(
    _topk_body,
    grid=(_B,),
    in_specs=[pl.BlockSpec((1, 1, _T), lambda b: (b, 0, 0))],
    out_specs=pl.BlockSpec((1, 1, _TPAD), lambda b: (b, 0, 0)),
    out_shape=jax.ShapeDtypeStruct((_B, 1, _TPAD), jnp.int32),
)


def _chunk_base(c):
    return c * _CHUNK if c < _NFULL else _K - _CHUNK


def _gather_body(x, rankp, shp, out, rank_v, idx_v, oidx_v, sh_v, buf0, buf1,
                 g0, g1, s0, s1):
    wid = lax.axis_index("s") * 2 + lax.axis_index("c")
    pltpu.sync_copy(rankp.at[wid], rank_v)
    pltpu.sync_copy(shp, sh_v)

    sh = sh_v[...]
    for k in range(_TPAD // _L):
        r16 = rank_v[0, pl.ds(k * _L, _L)]
        t16 = lax.iota(jnp.int32, _L) + (k * _L)
        tv = (t16 + sh) * _B + wid
        tok_ok = t16 <= _T - 1
        plsc.store_scatter(idx_v, [r16], tv,
                           mask=(r16 < _NFULL * _CHUNK) & tok_ok)
        plsc.store_scatter(idx_v, [r16 + (_NFULL + 1) * _CHUNK - _K], tv,
                           mask=(r16 >= _K - _CHUNK) & (r16 < _K) & tok_ok)

    nch = _NFULL + 1
    for c in range(nch):
        for v in range(_CHUNK // _L):
            p16 = lax.iota(jnp.int32, _L) + (_chunk_base(c) + v * _L)
            oidx_v[c, pl.ds(v * _L, _L)] = p16 * _B + wid

    bufs = (buf0, buf1)
    gsem = (g0, g1)
    ssem = (s0, s1)

    def start_gather(c):
        return pltpu.async_copy(x.at[idx_v.at[pl.ds(c * _CHUNK, _CHUNK)]],
                                bufs[c % 2], gsem[c % 2])

    def start_store(c):
        return pltpu.async_copy(bufs[c % 2], out.at[oidx_v.at[c]], ssem[c % 2])

    g_h = [None] * nch
    s_h = [None] * nch
    g_h[0] = start_gather(0)
    for c in range(nch):
        if c + 1 < nch:
            if c - 1 >= 0:
                s_h[c - 1].wait()
            g_h[c + 1] = start_gather(c + 1)
        g_h[c].wait()
        s_h[c] = start_store(c)
    s_h[nch - 2].wait()
    s_h[nch - 1].wait()


@functools.lru_cache(maxsize=1)
def _make_gather_call():
    return functools.partial(
        pl.kernel,
        out_type=jax.ShapeDtypeStruct((_B * _K, _C), jnp.float32),
        mesh=plsc.VectorSubcoreMesh(core_axis_name="c", subcore_axis_name="s"),
        compiler_params=pltpu.CompilerParams(needs_layout_passes=False),
        scratch_types=[
            pltpu.VMEM((1, _TPAD), jnp.int32),
            pltpu.VMEM(((_NFULL + 1) * _CHUNK,), jnp.int32),
            pltpu.VMEM((_NFULL + 1, _CHUNK), jnp.int32),
            pltpu.VMEM((_L,), jnp.int32),
            pltpu.VMEM((_CHUNK, _C), jnp.float32),
            pltpu.VMEM((_CHUNK, _C), jnp.float32),
            pltpu.SemaphoreType.DMA,
            pltpu.SemaphoreType.DMA,
            pltpu.SemaphoreType.DMA,
            pltpu.SemaphoreType.DMA,
        ],
    )(_gather_body)


def kernel(x, layer_idx, requested_r):
    del layer_idx
    xt = jnp.transpose(x, (1, 0, 2))
    st = _scores_call(xt).reshape(_T, _B)
    sb = jnp.transpose(st).reshape(_B, 1, _T)
    rankp = _topk_call(sb)
    shp = jnp.full((_L,), requested_r - 256, jnp.int32)
    xflat = xt.reshape(_T * _B, _C)
    outflat = _make_gather_call()(xflat, rankp, shp)
    return jnp.transpose(outflat.reshape(_K, _B, _C), (1, 0, 2))
